# baseline (speedup 1.0000x reference)
import jax
import jax.numpy as jnp
from jax.experimental import pallas as pl

_B, _F, _HID, _LAT, _K = 2048, 512, 128, 64, 1024
_BLK = 1024


def _gelu(v):
    return 0.5 * v * (1.0 + jax.lax.erf(v * jnp.float32(0.7071067811865476)))


def _split(a):
    hi = a.astype(jnp.bfloat16)
    lo = (a - hi.astype(jnp.float32)).astype(jnp.bfloat16)
    return hi, lo


def _mm_t(a, b):
    return jax.lax.dot_general(a, b, (((1,), (1,)), ((), ())),
                               preferred_element_type=jnp.float32)


def _mm_t3(a, b):
    a_hi, a_lo = _split(a)
    b_hi, b_lo = _split(b)
    a_cat = jnp.concatenate([a_hi, a_hi, a_lo], axis=1)
    b_cat = jnp.concatenate([b_hi, b_lo, b_hi], axis=1)
    return jax.lax.dot_general(a_cat, b_cat, (((1,), (1,)), ((), ())),
                               preferred_element_type=jnp.float32)


def _mm_t3_sep(a, b):
    a_hi, a_lo = _split(a)
    b_hi, b_lo = _split(b)
    f = lambda x, y: jax.lax.dot_general(
        x, y, (((1,), (1,)), ((), ())), preferred_element_type=jnp.float32)
    return (f(a_hi, b_hi) + f(a_hi, b_lo)) + f(a_lo, b_hi)


def _mm1(a_bf16, b_bf16):
    return jax.lax.dot_general(a_bf16, b_bf16, (((1,), (0,)), ((), ())),
                               preferred_element_type=jnp.float32)


def _mm_t1(a, b):
    return jax.lax.dot_general(a.astype(jnp.bfloat16), b.astype(jnp.bfloat16),
                               (((1,), (1,)), ((), ())),
                               preferred_element_type=jnp.float32)


def _fused_kernel(x_ref, w1_ref, b1_ref, w2_ref, b2_ref, lw_ref, lb_ref,
                  lci_ref, cb_ref, dw1_ref, db1_ref, dw2_ref, db2_ref,
                  ow_ref, ob_ref, zq_ref, loss_ref):
    i = pl.program_id(0)
    x = x_ref[...]
    h = _gelu(_mm_t3_sep(x, w1_ref[...]) + b1_ref[...])
    h = _gelu(_mm_t3(h, w2_ref[...]) + b2_ref[...])
    lw = lw_ref[...]
    absrowsum = jnp.sum(jnp.abs(lw), axis=1, keepdims=True)
    scale = jnp.minimum(jnp.float32(1.0),
                        jax.nn.softplus(lci_ref[...]) / absrowsum)
    wn = lw * scale
    z_e = jax.nn.sigmoid(_mm_t3(h, wn) + lb_ref[...])
    cb = cb_ref[...]
    z_hi, z_lo = _split(z_e)
    c_hi, c_lo = _split(cb)
    cn = jnp.sum(cb * cb, axis=1, keepdims=True)
    n_hi = cn.astype(jnp.bfloat16)
    r1 = cn - n_hi.astype(jnp.float32)
    n_lo = r1.astype(jnp.bfloat16)
    n_lo2 = (r1 - n_lo.astype(jnp.float32)).astype(jnp.bfloat16)
    a_cat = jnp.concatenate(
        [z_hi, z_hi, z_lo, jnp.ones((_BLK, 3), jnp.bfloat16)], axis=1)
    b_cat = jnp.concatenate(
        [-2.0 * c_hi, -2.0 * c_lo, -2.0 * c_hi, n_hi, n_lo, n_lo2], axis=1)
    scores = jax.lax.dot_general(a_cat, b_cat, (((1,), (1,)), ((), ())),
                                 preferred_element_type=jnp.float32)
    mins = jnp.min(scores, axis=1, keepdims=True)
    iota = jax.lax.broadcasted_iota(jnp.int32, (_BLK, _K), 1)
    idx = jnp.min(jnp.where(scores == mins, iota, _K), axis=1, keepdims=True)
    onehot = (iota == idx).astype(jnp.bfloat16)
    z_q = _mm1(onehot, c_hi)
    zq_ref[...] = z_q
    r = _gelu(_mm_t1(z_q, dw1_ref[...]) + db1_ref[...])
    r = _gelu(_mm_t1(r, dw2_ref[...]) + db2_ref[...])
    xr = _mm_t1(r, ow_ref[...]) + ob_ref[...]
    d = xr - x
    zd = z_q - z_e
    part = (jnp.sum(d * d) / jnp.float32(_B * _F)
            + 0.5 * jnp.sum(zd * zd) / jnp.float32(_B * _LAT))

    @pl.when(i == 0)
    def _init():
        loss_ref[...] = jnp.zeros_like(loss_ref)

    loss_ref[...] += part.reshape(1, 1)


@jax.jit
def kernel(x, enc_W1, enc_b1, enc_W2, enc_b2, lip_W, lip_b, lip_ci, codebook,
           dec_W1, dec_b1, dec_W2, dec_b2, out_W, out_b):
    full = lambda shape: pl.BlockSpec(shape, lambda i: (0, 0))
    z_q, loss = pl.pallas_call(
        _fused_kernel,
        grid=(_B // _BLK,),
        in_specs=[
            pl.BlockSpec((_BLK, _F), lambda i: (i, 0)),
            full((64, _F)), full((1, 64)),
            full((_HID, 64)), full((1, _HID)),
            full((_LAT, _HID)), full((1, _LAT)), full((_LAT, 1)),
            full((_K, _LAT)),
            full((64, _LAT)), full((1, 64)),
            full((_HID, 64)), full((1, _HID)),
            full((_F, _HID)), full((1, _F)),
        ],
        out_specs=[
            pl.BlockSpec((_BLK, _LAT), lambda i: (i, 0)),
            pl.BlockSpec((1, 1), lambda i: (0, 0)),
        ],
        out_shape=[
            jax.ShapeDtypeStruct((_B, _LAT), jnp.float32),
            jax.ShapeDtypeStruct((1, 1), jnp.float32),
        ],
    )(x, enc_W1, enc_b1.reshape(1, -1), enc_W2, enc_b2.reshape(1, -1),
      lip_W, lip_b.reshape(1, -1), lip_ci.reshape(-1, 1), codebook,
      dec_W1, dec_b1.reshape(1, -1), dec_W2, dec_b2.reshape(1, -1),
      out_W, out_b.reshape(1, -1))
    return z_q, loss[0, 0]


# speedup vs baseline: 1.0179x; 1.0179x over previous
import jax
import jax.numpy as jnp
from jax.experimental import pallas as pl

_B, _F, _HID, _LAT, _K = 2048, 512, 128, 64, 1024
_BLK = 2048
_HB = _BLK // 2


def _gelu(v):
    return 0.5 * v * (1.0 + jax.lax.erf(v * jnp.float32(0.7071067811865476)))


def _split(a):
    hi = a.astype(jnp.bfloat16)
    lo = (a - hi.astype(jnp.float32)).astype(jnp.bfloat16)
    return hi, lo


def _mm_t(a, b):
    return jax.lax.dot_general(a, b, (((1,), (1,)), ((), ())),
                               preferred_element_type=jnp.float32)


def _mm_t3(a, b):
    a_hi, a_lo = _split(a)
    b_hi, b_lo = _split(b)
    a_cat = jnp.concatenate([a_hi, a_hi, a_lo], axis=1)
    b_cat = jnp.concatenate([b_hi, b_lo, b_hi], axis=1)
    return jax.lax.dot_general(a_cat, b_cat, (((1,), (1,)), ((), ())),
                               preferred_element_type=jnp.float32)


def _mm_t3_sep(a, b):
    a_hi, a_lo = _split(a)
    b_hi, b_lo = _split(b)
    f = lambda x, y: jax.lax.dot_general(
        x, y, (((1,), (1,)), ((), ())), preferred_element_type=jnp.float32)
    return (f(a_hi, b_hi) + f(a_hi, b_lo)) + f(a_lo, b_hi)


def _mm1(a_bf16, b_bf16):
    return jax.lax.dot_general(a_bf16, b_bf16, (((1,), (0,)), ((), ())),
                               preferred_element_type=jnp.float32)


def _mm_t1(a, b):
    return jax.lax.dot_general(a.astype(jnp.bfloat16), b.astype(jnp.bfloat16),
                               (((1,), (1,)), ((), ())),
                               preferred_element_type=jnp.float32)


def _fused_kernel(x_ref, w1_ref, b1_ref, w2_ref, b2_ref, lw_ref, lb_ref,
                  lci_ref, cb_ref, dw1_ref, db1_ref, dw2_ref, db2_ref,
                  ow_ref, ob_ref, zq_ref, loss_ref):
    i = pl.program_id(0)
    lw = lw_ref[...]
    absrowsum = jnp.sum(jnp.abs(lw), axis=1, keepdims=True)
    scale = jnp.minimum(jnp.float32(1.0),
                        jax.nn.softplus(lci_ref[...]) / absrowsum)
    wn = lw * scale
    cb = cb_ref[...]
    c_hi, c_lo = _split(cb)
    cn = jnp.sum(cb * cb, axis=1, keepdims=True)
    n_hi = cn.astype(jnp.bfloat16)
    r1 = cn - n_hi.astype(jnp.float32)
    n_lo = r1.astype(jnp.bfloat16)
    n_lo2 = (r1 - n_lo.astype(jnp.float32)).astype(jnp.bfloat16)
    b_cat = jnp.concatenate(
        [-2.0 * c_hi, -2.0 * c_lo, -2.0 * c_hi, n_hi, n_lo, n_lo2], axis=1)

    def _half(x):
        h = _gelu(_mm_t3_sep(x, w1_ref[...]) + b1_ref[...])
        h = _gelu(_mm_t3(h, w2_ref[...]) + b2_ref[...])
        z_e = jax.nn.sigmoid(_mm_t3(h, wn) + lb_ref[...])
        z_hi, z_lo = _split(z_e)
        a_cat = jnp.concatenate(
            [z_hi, z_hi, z_lo, jnp.ones((_HB, 3), jnp.bfloat16)], axis=1)
        scores = jax.lax.dot_general(a_cat, b_cat, (((1,), (1,)), ((), ())),
                                     preferred_element_type=jnp.float32)
        mins = jnp.min(scores, axis=1, keepdims=True)
        iota = jax.lax.broadcasted_iota(jnp.int32, (_HB, _K), 1)
        idx = jnp.min(jnp.where(scores == mins, iota, _K), axis=1,
                      keepdims=True)
        onehot = (iota == idx).astype(jnp.bfloat16)
        z_q = _mm1(onehot, c_hi)
        r = _gelu(_mm_t1(z_q, dw1_ref[...]) + db1_ref[...])
        r = _gelu(_mm_t1(r, dw2_ref[...]) + db2_ref[...])
        xr = _mm_t1(r, ow_ref[...]) + ob_ref[...]
        d = xr - x
        zd = z_q - z_e
        part = (jnp.sum(d * d) / jnp.float32(_B * _F)
                + 0.5 * jnp.sum(zd * zd) / jnp.float32(_B * _LAT))
        return z_q, part

    zq0, p0 = _half(x_ref[0:_HB, :])
    zq1, p1 = _half(x_ref[_HB:_BLK, :])
    zq_ref[0:_HB, :] = zq0
    zq_ref[_HB:_BLK, :] = zq1
    part = p0 + p1
    @pl.when(i == 0)
    def _init():
        loss_ref[...] = jnp.zeros_like(loss_ref)

    loss_ref[...] += part.reshape(1, 1)


@jax.jit
def kernel(x, enc_W1, enc_b1, enc_W2, enc_b2, lip_W, lip_b, lip_ci, codebook,
           dec_W1, dec_b1, dec_W2, dec_b2, out_W, out_b):
    full = lambda shape: pl.BlockSpec(shape, lambda i: (0, 0))
    z_q, loss = pl.pallas_call(
        _fused_kernel,
        grid=(_B // _BLK,),
        in_specs=[
            pl.BlockSpec((_BLK, _F), lambda i: (i, 0)),
            full((64, _F)), full((1, 64)),
            full((_HID, 64)), full((1, _HID)),
            full((_LAT, _HID)), full((1, _LAT)), full((_LAT, 1)),
            full((_K, _LAT)),
            full((64, _LAT)), full((1, 64)),
            full((_HID, 64)), full((1, _HID)),
            full((_F, _HID)), full((1, _F)),
        ],
        out_specs=[
            pl.BlockSpec((_BLK, _LAT), lambda i: (i, 0)),
            pl.BlockSpec((1, 1), lambda i: (0, 0)),
        ],
        out_shape=[
            jax.ShapeDtypeStruct((_B, _LAT), jnp.float32),
            jax.ShapeDtypeStruct((1, 1), jnp.float32),
        ],
    )(x, enc_W1, enc_b1.reshape(1, -1), enc_W2, enc_b2.reshape(1, -1),
      lip_W, lip_b.reshape(1, -1), lip_ci.reshape(-1, 1), codebook,
      dec_W1, dec_b1.reshape(1, -1), dec_W2, dec_b2.reshape(1, -1),
      out_W, out_b.reshape(1, -1))
    return z_q, loss[0, 0]


# speedup vs baseline: 1.0397x; 1.0214x over previous
import jax
import jax.numpy as jnp
from jax.experimental import pallas as pl

_B, _F, _HID, _LAT, _K = 2048, 512, 128, 64, 1024
_BLK = 2048


def _gelu(v):
    return 0.5 * v * (1.0 + jax.lax.erf(v * jnp.float32(0.7071067811865476)))


def _split(a):
    hi = a.astype(jnp.bfloat16)
    lo = (a - hi.astype(jnp.float32)).astype(jnp.bfloat16)
    return hi, lo


def _mm_t(a, b):
    return jax.lax.dot_general(a, b, (((1,), (1,)), ((), ())),
                               preferred_element_type=jnp.float32)


def _mm_t3(a, b):
    a_hi, a_lo = _split(a)
    b_hi, b_lo = _split(b)
    a_cat = jnp.concatenate([a_hi, a_hi, a_lo], axis=1)
    b_cat = jnp.concatenate([b_hi, b_lo, b_hi], axis=1)
    return jax.lax.dot_general(a_cat, b_cat, (((1,), (1,)), ((), ())),
                               preferred_element_type=jnp.float32)


def _mm_t3_sep(a, b):
    a_hi, a_lo = _split(a)
    b_hi, b_lo = _split(b)
    f = lambda x, y: jax.lax.dot_general(
        x, y, (((1,), (1,)), ((), ())), preferred_element_type=jnp.float32)
    return (f(a_hi, b_hi) + f(a_hi, b_lo)) + f(a_lo, b_hi)


def _mm1(a_bf16, b_bf16):
    return jax.lax.dot_general(a_bf16, b_bf16, (((1,), (0,)), ((), ())),
                               preferred_element_type=jnp.float32)


def _mm_t1(a, b):
    return jax.lax.dot_general(a.astype(jnp.bfloat16), b.astype(jnp.bfloat16),
                               (((1,), (1,)), ((), ())),
                               preferred_element_type=jnp.float32)


def _fused_kernel(x_ref, w1_ref, b1_ref, w2_ref, b2_ref, lw_ref, lb_ref,
                  lci_ref, cb_ref, dw1_ref, db1_ref, dw2_ref, db2_ref,
                  ow_ref, ob_ref, zq_ref, loss_ref):
    x = x_ref[...]
    h = _gelu(_mm_t3_sep(x, w1_ref[...]) + b1_ref[...])
    h = _gelu(_mm_t3(h, w2_ref[...]) + b2_ref[...])
    lw = lw_ref[...]
    absrowsum = jnp.sum(jnp.abs(lw), axis=1, keepdims=True)
    scale = jnp.minimum(jnp.float32(1.0),
                        jax.nn.softplus(lci_ref[...]) / absrowsum)
    wn = lw * scale
    z_e = jax.nn.sigmoid(_mm_t3(h, wn) + lb_ref[...])
    cb = cb_ref[...]
    z_hi, z_lo = _split(z_e)
    c_hi, c_lo = _split(cb)
    cn = jnp.sum(cb * cb, axis=1, keepdims=True)
    n_hi = cn.astype(jnp.bfloat16)
    r1 = cn - n_hi.astype(jnp.float32)
    n_lo = r1.astype(jnp.bfloat16)
    n_lo2 = (r1 - n_lo.astype(jnp.float32)).astype(jnp.bfloat16)
    a_cat = jnp.concatenate(
        [z_hi, z_hi, z_lo, jnp.ones((_BLK, 3), jnp.bfloat16)], axis=1)
    b_cat = jnp.concatenate(
        [-2.0 * c_hi, -2.0 * c_lo, -2.0 * c_hi, n_hi, n_lo, n_lo2], axis=1)
    scores = jax.lax.dot_general(a_cat, b_cat, (((1,), (1,)), ((), ())),
                                 preferred_element_type=jnp.float32)
    mins = jnp.min(scores, axis=1, keepdims=True)
    iota = jax.lax.broadcasted_iota(jnp.int32, (_BLK, _K), 1)
    idx = jnp.min(jnp.where(scores == mins, iota, _K), axis=1, keepdims=True)
    onehot = (iota == idx).astype(jnp.bfloat16)
    z_q = _mm1(onehot, c_hi)
    zq_ref[...] = z_q
    r = _gelu(_mm_t1(z_q, dw1_ref[...]) + db1_ref[...])
    r = _gelu(_mm_t1(r, dw2_ref[...]) + db2_ref[...])
    xr = _mm_t1(r, ow_ref[...]) + ob_ref[...]
    d = xr - x
    zd = z_q - z_e
    part = (jnp.sum(d * d) / jnp.float32(_B * _F)
            + 0.5 * jnp.sum(zd * zd) / jnp.float32(_B * _LAT))
    loss_ref[...] = part.reshape(1, 1)


@jax.jit
def kernel(x, enc_W1, enc_b1, enc_W2, enc_b2, lip_W, lip_b, lip_ci, codebook,
           dec_W1, dec_b1, dec_W2, dec_b2, out_W, out_b):
    full = lambda shape: pl.BlockSpec(shape, lambda i: (0, 0))
    z_q, loss = pl.pallas_call(
        _fused_kernel,
        grid=(_B // _BLK,),
        in_specs=[
            pl.BlockSpec((_BLK, _F), lambda i: (i, 0)),
            full((64, _F)), full((1, 64)),
            full((_HID, 64)), full((1, _HID)),
            full((_LAT, _HID)), full((1, _LAT)), full((_LAT, 1)),
            full((_K, _LAT)),
            full((64, _LAT)), full((1, 64)),
            full((_HID, 64)), full((1, _HID)),
            full((_F, _HID)), full((1, _F)),
        ],
        out_specs=[
            pl.BlockSpec((_BLK, _LAT), lambda i: (i, 0)),
            pl.BlockSpec((1, 1), lambda i: (0, 0)),
        ],
        out_shape=[
            jax.ShapeDtypeStruct((_B, _LAT), jnp.float32),
            jax.ShapeDtypeStruct((1, 1), jnp.float32),
        ],
    )(x, enc_W1, enc_b1.reshape(1, -1), enc_W2, enc_b2.reshape(1, -1),
      lip_W, lip_b.reshape(1, -1), lip_ci.reshape(-1, 1), codebook,
      dec_W1, dec_b1.reshape(1, -1), dec_W2, dec_b2.reshape(1, -1),
      out_W, out_b.reshape(1, -1))
    return z_q, loss[0, 0]


# speedup vs baseline: 1.1112x; 1.0688x over previous
import jax
import jax.numpy as jnp
from jax.experimental import pallas as pl

_B, _F, _HID, _LAT, _K = 2048, 512, 128, 64, 1024
_BLK = 2048


def _gelu(v):
    return 0.5 * v * (1.0 + jax.lax.erf(v * jnp.float32(0.7071067811865476)))


def _split(a):
    hi = a.astype(jnp.bfloat16)
    lo = (a - hi.astype(jnp.float32)).astype(jnp.bfloat16)
    return hi, lo


def _mm_t(a, b):
    return jax.lax.dot_general(a, b, (((1,), (1,)), ((), ())),
                               preferred_element_type=jnp.float32)


def _mm_t3(a, b):
    a_hi, a_lo = _split(a)
    b_hi, b_lo = _split(b)
    a_cat = jnp.concatenate([a_hi, a_hi, a_lo], axis=1)
    b_cat = jnp.concatenate([b_hi, b_lo, b_hi], axis=1)
    return jax.lax.dot_general(a_cat, b_cat, (((1,), (1,)), ((), ())),
                               preferred_element_type=jnp.float32)


def _mm_t3_sep(a, b):
    a_hi, a_lo = _split(a)
    b_hi, b_lo = _split(b)
    f = lambda x, y: jax.lax.dot_general(
        x, y, (((1,), (1,)), ((), ())), preferred_element_type=jnp.float32)
    return (f(a_hi, b_hi) + f(a_hi, b_lo)) + f(a_lo, b_hi)


def _mm1(a_bf16, b_bf16):
    return jax.lax.dot_general(a_bf16, b_bf16, (((1,), (0,)), ((), ())),
                               preferred_element_type=jnp.float32)


def _mm_t1(a, b):
    return jax.lax.dot_general(a.astype(jnp.bfloat16), b.astype(jnp.bfloat16),
                               (((1,), (1,)), ((), ())),
                               preferred_element_type=jnp.float32)


def _fused_kernel(x_ref, w1_ref, b1_ref, w2_ref, b2_ref, lw_ref, lb_ref,
                  lci_ref, cb_ref, dw1_ref, db1_ref, dw2_ref, db2_ref,
                  ow_ref, ob_ref, zq_ref, loss_ref):
    x = x_ref[...]
    h = _gelu(_mm_t1(x, w1_ref[...]) + b1_ref[...])
    h = _gelu(_mm_t1(h, w2_ref[...]) + b2_ref[...])
    lw = lw_ref[...]
    absrowsum = jnp.sum(jnp.abs(lw), axis=1, keepdims=True)
    scale = jnp.minimum(jnp.float32(1.0),
                        jax.nn.softplus(lci_ref[...]) / absrowsum)
    wn = lw * scale
    z_e = jax.nn.sigmoid(_mm_t1(h, wn) + lb_ref[...])
    cb = cb_ref[...]
    z_hi, z_lo = _split(z_e)
    c_hi, c_lo = _split(cb)
    cn = jnp.sum(cb * cb, axis=1, keepdims=True)
    n_hi = cn.astype(jnp.bfloat16)
    r1 = cn - n_hi.astype(jnp.float32)
    n_lo = r1.astype(jnp.bfloat16)
    n_lo2 = (r1 - n_lo.astype(jnp.float32)).astype(jnp.bfloat16)
    a_cat = jnp.concatenate(
        [z_hi, z_hi, z_lo, jnp.ones((_BLK, 3), jnp.bfloat16)], axis=1)
    b_cat = jnp.concatenate(
        [-2.0 * c_hi, -2.0 * c_lo, -2.0 * c_hi, n_hi, n_lo, n_lo2], axis=1)
    scores = jax.lax.dot_general(a_cat, b_cat, (((1,), (1,)), ((), ())),
                                 preferred_element_type=jnp.float32)
    mins = jnp.min(scores, axis=1, keepdims=True)
    iota = jax.lax.broadcasted_iota(jnp.int32, (_BLK, _K), 1)
    idx = jnp.min(jnp.where(scores == mins, iota, _K), axis=1, keepdims=True)
    onehot = (iota == idx).astype(jnp.bfloat16)
    z_q = _mm1(onehot, c_hi)
    zq_ref[...] = z_q
    r = _gelu(_mm_t1(z_q, dw1_ref[...]) + db1_ref[...])
    r = _gelu(_mm_t1(r, dw2_ref[...]) + db2_ref[...])
    xr = _mm_t1(r, ow_ref[...]) + ob_ref[...]
    d = xr - x
    zd = z_q - z_e
    part = (jnp.sum(d * d) / jnp.float32(_B * _F)
            + 0.5 * jnp.sum(zd * zd) / jnp.float32(_B * _LAT))
    loss_ref[...] = part.reshape(1, 1)


@jax.jit
def kernel(x, enc_W1, enc_b1, enc_W2, enc_b2, lip_W, lip_b, lip_ci, codebook,
           dec_W1, dec_b1, dec_W2, dec_b2, out_W, out_b):
    full = lambda shape: pl.BlockSpec(shape, lambda i: (0, 0))
    z_q, loss = pl.pallas_call(
        _fused_kernel,
        grid=(_B // _BLK,),
        in_specs=[
            pl.BlockSpec((_BLK, _F), lambda i: (i, 0)),
            full((64, _F)), full((1, 64)),
            full((_HID, 64)), full((1, _HID)),
            full((_LAT, _HID)), full((1, _LAT)), full((_LAT, 1)),
            full((_K, _LAT)),
            full((64, _LAT)), full((1, 64)),
            full((_HID, 64)), full((1, _HID)),
            full((_F, _HID)), full((1, _F)),
        ],
        out_specs=[
            pl.BlockSpec((_BLK, _LAT), lambda i: (i, 0)),
            pl.BlockSpec((1, 1), lambda i: (0, 0)),
        ],
        out_shape=[
            jax.ShapeDtypeStruct((_B, _LAT), jnp.float32),
            jax.ShapeDtypeStruct((1, 1), jnp.float32),
        ],
    )(x, enc_W1, enc_b1.reshape(1, -1), enc_W2, enc_b2.reshape(1, -1),
      lip_W, lip_b.reshape(1, -1), lip_ci.reshape(-1, 1), codebook,
      dec_W1, dec_b1.reshape(1, -1), dec_W2, dec_b2.reshape(1, -1),
      out_W, out_b.reshape(1, -1))
    return z_q, loss[0, 0]
